# Initial kernel scaffold; baseline (speedup 1.0000x reference)
import jax
import jax.numpy as jnp
from jax.experimental import pallas as pl


def kernel(x_omics1, x_omics2, edge_index, params):
    raise NotImplementedError("write your pallas kernel here")


# speedup vs baseline: 5.4733x; 5.4733x over previous
import functools

import jax
import jax.numpy as jnp
from jax import lax
from jax.experimental import pallas as pl
from jax.experimental.pallas import tpu as pltpu
from jax.experimental.pallas import tpu_sc as plsc

N = 10000
NPAD = 10240
SINK = 10000
NW = 32
CH = 128
NSUB = 16
ZR = NPAD // NSUB
ZDIM = 32
DEGW = 16

_f32 = jnp.float32

@functools.cache
def _sc_mesh():
    return plsc.VectorSubcoreMesh(core_axis_name="c", subcore_axis_name="s")


_linear_cp = pltpu.CompilerParams(use_tc_tiling_on_sc=False)


def _sc_degree(dstw):
    nch = dstw.shape[1]

    @functools.partial(
        pl.kernel, mesh=_sc_mesh(), compiler_params=_linear_cp,
        out_type=jax.ShapeDtypeStruct((2, NPAD, DEGW), _f32),
        scratch_types=[
            pltpu.VMEM((nch, CH), jnp.int32),
            pltpu.VMEM((CH, DEGW), _f32),
            pltpu.VMEM_SHARED((NPAD, DEGW), _f32),
        ],
    )
    def k(dstw_hbm, zeros_hbm, ones_hbm, out_hbm, didx, ones_v, acc):
        cid = lax.axis_index("c")
        sid = lax.axis_index("s")
        wid = sid * 2 + cid
        pltpu.sync_copy(zeros_hbm.at[pl.ds(0, ZR)], acc.at[pl.ds(sid * ZR, ZR)])
        pltpu.sync_copy(dstw_hbm.at[wid], didx)
        pltpu.sync_copy(ones_hbm, ones_v)
        plsc.subcore_barrier()

        @pl.loop(0, nch)
        def _(j):
            pltpu.sync_copy(ones_v, acc.at[didx.at[j]], add=True)

        plsc.subcore_barrier()
        pltpu.sync_copy(acc.at[pl.ds(sid * ZR, ZR)],
                        out_hbm.at[cid, pl.ds(sid * ZR, ZR)])

    zeros = jnp.zeros((ZR, DEGW), _f32)
    ones = jnp.ones((CH, DEGW), _f32)
    return k(dstw, zeros, ones)


def _sc_prop(xs, srcw, dstw):
    nch = srcw.shape[1]
    f = xs.shape[1]

    @functools.partial(
        pl.kernel, mesh=_sc_mesh(),
        out_type=jax.ShapeDtypeStruct((2, NPAD, f), _f32),
        scratch_types=[
            pltpu.VMEM((nch, CH), jnp.int32),
            pltpu.VMEM((nch, CH), jnp.int32),
            pltpu.VMEM((CH, f), _f32),
            pltpu.VMEM_SHARED((NPAD, f), _f32),
        ],
    )
    def k(xs_hbm, srcw_hbm, dstw_hbm, zeros_hbm, out_hbm,
          sidx, didx, rows, acc):
        cid = lax.axis_index("c")
        sid = lax.axis_index("s")
        wid = sid * 2 + cid
        pltpu.sync_copy(zeros_hbm.at[pl.ds(0, ZR)], acc.at[pl.ds(sid * ZR, ZR)])
        pltpu.sync_copy(srcw_hbm.at[wid], sidx)
        pltpu.sync_copy(dstw_hbm.at[wid], didx)
        plsc.subcore_barrier()

        @pl.loop(0, nch)
        def _(j):
            pltpu.sync_copy(xs_hbm.at[sidx.at[j]], rows)
            pltpu.sync_copy(rows, acc.at[didx.at[j]], add=True)

        plsc.subcore_barrier()
        pltpu.sync_copy(acc.at[pl.ds(sid * ZR, ZR)],
                        out_hbm.at[cid, pl.ds(sid * ZR, ZR)])

    zeros = jnp.zeros((ZR, f), _f32)
    return k(xs, srcw, dstw, zeros)


def _sc_zgather(z, srcw, dstw):
    nch = srcw.shape[1]
    epad = NW * nch * CH
    epw = nch * CH
    out_sds = jax.ShapeDtypeStruct((epad, ZDIM), _f32)

    @functools.partial(
        pl.kernel, mesh=_sc_mesh(), compiler_params=_linear_cp,
        out_type=(out_sds, out_sds),
        scratch_types=[
            pltpu.VMEM((nch, CH), jnp.int32),
            pltpu.VMEM((nch, CH), jnp.int32),
            pltpu.VMEM((CH, ZDIM), _f32),
        ],
    )
    def k(z_hbm, srcw_hbm, dstw_hbm, zs_hbm, zd_hbm, sidx, didx, buf):
        cid = lax.axis_index("c")
        sid = lax.axis_index("s")
        wid = sid * 2 + cid
        pltpu.sync_copy(srcw_hbm.at[wid], sidx)
        pltpu.sync_copy(dstw_hbm.at[wid], didx)

        @pl.loop(0, nch)
        def _(j):
            base = wid * epw + j * CH
            pltpu.sync_copy(z_hbm.at[sidx.at[j]], buf)
            pltpu.sync_copy(buf, zs_hbm.at[pl.ds(base, CH)])
            pltpu.sync_copy(z_hbm.at[didx.at[j]], buf)
            pltpu.sync_copy(buf, zd_hbm.at[pl.ds(base, CH)])

    return k(z, srcw, dstw)


_RB = 1000


def _full(shape):
    return pl.BlockSpec(shape, lambda i: tuple(0 for _ in shape))


def _rows(shape):
    return pl.BlockSpec(shape, lambda i: (i,) + tuple(0 for _ in shape[1:]))


def _rows3(shape):
    return pl.BlockSpec(shape, lambda i: (0, i, 0))


def _tc_a0(x1, x2, w1, w2):
    def body(x1_ref, x2_ref, w1_ref, w2_ref, o1_ref, o2_ref):
        o1_ref[...] = jnp.dot(x1_ref[...], w1_ref[...],
                              preferred_element_type=_f32)
        o2_ref[...] = jnp.dot(x2_ref[...], w2_ref[...],
                              preferred_element_type=_f32)

    d = x1.shape[1]
    h = w1.shape[1]
    return pl.pallas_call(
        body,
        grid=(N // _RB,),
        in_specs=[_rows((_RB, d)), _rows((_RB, d)), _full((d, h)), _full((d, h))],
        out_specs=[_rows((_RB, h)), _rows((_RB, h))],
        out_shape=[jax.ShapeDtypeStruct((N, h), _f32)] * 2,
    )(x1, x2, w1, w2)


def _tc_a1(degp, xw1, xw2):
    def body(degp_ref, xw1_ref, xw2_ref, xs1_ref, xs2_ref, dinv_ref):
        dp = degp_ref[...]
        deg = dp[0] + dp[1] + 1.0
        dinv = 1.0 / jnp.sqrt(deg)
        dinv_ref[...] = dinv
        d1 = dinv[:, 0:1]
        xs1_ref[...] = xw1_ref[...] * d1
        xs2_ref[...] = xw2_ref[...] * d1

    h = xw1.shape[1]
    return pl.pallas_call(
        body,
        grid=(N // _RB,),
        in_specs=[_rows3((2, _RB, DEGW)), _rows((_RB, h)), _rows((_RB, h))],
        out_specs=[_rows((_RB, h)), _rows((_RB, h)), _rows((_RB, DEGW))],
        out_shape=[jax.ShapeDtypeStruct((N, h), _f32),
                   jax.ShapeDtypeStruct((N, h), _f32),
                   jax.ShapeDtypeStruct((N, DEGW), _f32)],
    )(degp, xw1, xw2)


def _tc_b(acc1, acc2, xs1, xs2, dinv16, w2a, w2b, b1a, b1b):
    def body(a1_ref, a2_ref, xs1_ref, xs2_ref, dinv_ref,
             w2a_ref, w2b_ref, b1a_ref, b1b_ref, out_ref):
        dinv = dinv_ref[:, 0:1]
        a1 = a1_ref[...]
        a2 = a2_ref[...]
        h1 = jax.nn.relu(dinv * (a1[0] + a1[1] + xs1_ref[...]) + b1a_ref[...])
        h2 = jax.nn.relu(dinv * (a2[0] + a2[1] + xs2_ref[...]) + b1b_ref[...])
        xw1 = jnp.dot(h1, w2a_ref[...], preferred_element_type=_f32)
        xw2 = jnp.dot(h2, w2b_ref[...], preferred_element_type=_f32)
        out_ref[...] = jnp.concatenate([xw1 * dinv, xw2 * dinv], axis=1)

    h1w = xs1.shape[1]
    h2w = w2a.shape[1]
    return pl.pallas_call(
        body,
        grid=(N // _RB,),
        in_specs=[_rows3((2, _RB, h1w)), _rows3((2, _RB, h1w)),
                  _rows((_RB, h1w)), _rows((_RB, h1w)), _rows((_RB, DEGW)),
                  _full((h1w, h2w)), _full((h1w, h2w)),
                  _full((1, h1w)), _full((1, h1w))],
        out_specs=[_rows((_RB, 2 * h2w))],
        out_shape=[jax.ShapeDtypeStruct((N, 2 * h2w), _f32)],
    )(acc1, acc2, xs1, xs2, dinv16, w2a, w2b, b1a, b1b)[0]


def _tc_c1(acc3, xscat, dinv16, b2cat, fw, fb, muw, mub, lvw, lvb, eps):
    def body(a_ref, xs_ref, dinv_ref, b2_ref, fw_ref, fb_ref,
             muw_ref, mub_ref, lvw_ref, lvb_ref, eps_ref,
             z_ref, mu_ref, lv_ref):
        dinv = dinv_ref[:, 0:1]
        a = a_ref[...]
        hcat = jax.nn.relu(dinv * (a[0] + a[1] + xs_ref[...]) + b2_ref[...])
        h = jax.nn.relu(jnp.dot(hcat, fw_ref[...],
                                preferred_element_type=_f32) + fb_ref[...])
        mu = jnp.dot(h, muw_ref[...], preferred_element_type=_f32) + mub_ref[...]
        lv = jnp.dot(h, lvw_ref[...], preferred_element_type=_f32) + lvb_ref[...]
        mu_ref[...] = mu
        lv_ref[...] = lv
        z_ref[...] = mu + eps_ref[...] * jnp.exp(0.5 * lv)

    cw = xscat.shape[1]
    fu = fw.shape[1]
    return pl.pallas_call(
        body,
        grid=(N // _RB,),
        in_specs=[_rows3((2, _RB, cw)), _rows((_RB, cw)), _rows((_RB, DEGW)),
                  _full((1, cw)), _full((cw, fu)), _full((1, fu)),
                  _full((fu, ZDIM)), _full((1, ZDIM)),
                  _full((fu, ZDIM)), _full((1, ZDIM)),
                  _rows((_RB, ZDIM))],
        out_specs=[_rows((_RB, ZDIM))] * 3,
        out_shape=[jax.ShapeDtypeStruct((N, ZDIM), _f32)] * 3,
    )(acc3, xscat, dinv16, b2cat, fw, fb, muw, mub, lvw, lvb, eps)


def _tc_c2(z, w11, b11, w12, b12, w21, b21, w22, b22, ws1, bs1, ws2, bs2):
    def body(z_ref, w11_ref, b11_ref, w12_ref, b12_ref,
             w21_ref, b21_ref, w22_ref, b22_ref,
             ws1_ref, bs1_ref, ws2_ref, bs2_ref,
             x1_ref, x2_ref, xs_ref):
        z = z_ref[...]
        t1 = jax.nn.relu(jnp.dot(z, w11_ref[...],
                                 preferred_element_type=_f32) + b11_ref[...])
        x1_ref[...] = jnp.dot(t1, w12_ref[...],
                              preferred_element_type=_f32) + b12_ref[...]
        t2 = jax.nn.relu(jnp.dot(z, w21_ref[...],
                                 preferred_element_type=_f32) + b21_ref[...])
        x2_ref[...] = jnp.dot(t2, w22_ref[...],
                              preferred_element_type=_f32) + b22_ref[...]
        ts = jax.nn.relu(jnp.dot(z, ws1_ref[...],
                                 preferred_element_type=_f32) + bs1_ref[...])
        xs_ref[...] = jnp.dot(ts, ws2_ref[...],
                              preferred_element_type=_f32) + bs2_ref[...]

    fu = w11.shape[1]
    r1 = w12.shape[1]
    r2 = w22.shape[1]
    rs = ws2.shape[1]
    return pl.pallas_call(
        body,
        grid=(N // _RB,),
        in_specs=[_rows((_RB, ZDIM)),
                  _full((ZDIM, fu)), _full((1, fu)), _full((fu, r1)), _full((1, r1)),
                  _full((ZDIM, fu)), _full((1, fu)), _full((fu, r2)), _full((1, r2)),
                  _full((ZDIM, fu)), _full((1, fu)), _full((fu, rs)), _full((1, rs))],
        out_specs=[_rows((_RB, r1)), _rows((_RB, r2)), _rows((_RB, rs))],
        out_shape=[jax.ShapeDtypeStruct((N, r1), _f32),
                   jax.ShapeDtypeStruct((N, r2), _f32),
                   jax.ShapeDtypeStruct((N, rs), _f32)],
    )(z, w11, b11, w12, b12, w21, b21, w22, b22, ws1, bs1, ws2, bs2)


def _tc_d(zs, zd):
    eb = 2048

    def body(zs_ref, zd_ref, o_ref):
        prod = zs_ref[...] * zd_ref[...]
        s = jnp.dot(prod, jnp.ones((ZDIM, 1), _f32),
                    preferred_element_type=_f32)
        o_ref[...] = jax.nn.sigmoid(s)

    epad = zs.shape[0]
    return pl.pallas_call(
        body,
        grid=(epad // eb,),
        in_specs=[_rows((eb, ZDIM)), _rows((eb, ZDIM))],
        out_specs=[_rows((eb, 1))],
        out_shape=[jax.ShapeDtypeStruct((epad, 1), _f32)],
    )(zs, zd)[0]


def kernel(x_omics1, x_omics2, edge_index, params):
    p = params
    src = edge_index[0].astype(jnp.int32)
    dst = edge_index[1].astype(jnp.int32)
    e = src.shape[0]
    epad = -(-e // (NW * CH)) * (NW * CH)
    nch = epad // (NW * CH)
    pad = epad - e
    srcw = jnp.concatenate([src, jnp.zeros((pad,), jnp.int32)]).reshape(NW, nch, CH)
    dstw_sink = jnp.concatenate(
        [dst, jnp.full((pad,), SINK, jnp.int32)]).reshape(NW, nch, CH)
    dstw_zero = jnp.concatenate(
        [dst, jnp.zeros((pad,), jnp.int32)]).reshape(NW, nch, CH)

    degp = _sc_degree(dstw_sink)
    xw1, xw2 = _tc_a0(x_omics1, x_omics2, p['gcn1_w1'], p['gcn2_w1'])
    xs1, xs2, dinv16 = _tc_a1(degp, xw1, xw2)

    acc1 = _sc_prop(xs1, srcw, dstw_sink)
    acc2 = _sc_prop(xs2, srcw, dstw_sink)

    xscat = _tc_b(acc1, acc2, xs1, xs2, dinv16,
                  p['gcn1_w2'], p['gcn2_w2'],
                  p['gcn1_b1'][None, :], p['gcn2_b1'][None, :])

    acc3 = _sc_prop(xscat, srcw, dstw_sink)

    b2cat = jnp.concatenate([p['gcn1_b2'], p['gcn2_b2']])[None, :]
    eps = jax.random.normal(jax.random.key(42), (N, ZDIM), _f32)
    z, mu, logvar = _tc_c1(acc3, xscat, dinv16, b2cat,
                           p['fuse_w'], p['fuse_b'][None, :],
                           p['mu_w'], p['mu_b'][None, :],
                           p['logvar_w'], p['logvar_b'][None, :], eps)

    xhat1, xhat2, xhat_s = _tc_c2(
        z, p['rec1_w1'], p['rec1_b1'][None, :], p['rec1_w2'], p['rec1_b2'][None, :],
        p['rec2_w1'], p['rec2_b1'][None, :], p['rec2_w2'], p['rec2_b2'][None, :],
        p['recs_w1'], p['recs_b1'][None, :], p['recs_w2'], p['recs_b2'][None, :])

    zs, zd = _sc_zgather(z, srcw, dstw_zero)
    adj = _tc_d(zs, zd)
    adj_pred = adj[:e, 0]

    return (z, mu, logvar, xhat1, xhat2, xhat_s, adj_pred)


# speedup vs baseline: 5.9796x; 1.0925x over previous
import functools

import jax
import jax.numpy as jnp
from jax import lax
from jax.experimental import pallas as pl
from jax.experimental.pallas import tpu as pltpu
from jax.experimental.pallas import tpu_sc as plsc

N = 10000
NPAD = 10240
SINK = 10000
NW = 32
CH = 128
NSUB = 16
ZR = NPAD // NSUB
ZDIM = 32
DEGW = 16

_f32 = jnp.float32

@functools.cache
def _sc_mesh():
    return plsc.VectorSubcoreMesh(core_axis_name="c", subcore_axis_name="s")


_linear_cp = pltpu.CompilerParams(use_tc_tiling_on_sc=False)


def _sc_degree(dstw):
    nch = dstw.shape[1]

    @functools.partial(
        pl.kernel, mesh=_sc_mesh(), compiler_params=_linear_cp,
        out_type=jax.ShapeDtypeStruct((2, NPAD, DEGW), _f32),
        scratch_types=[
            pltpu.VMEM((nch, CH), jnp.int32),
            pltpu.VMEM((CH, DEGW), _f32),
            pltpu.VMEM_SHARED((NPAD, DEGW), _f32),
        ],
    )
    def k(dstw_hbm, zeros_hbm, ones_hbm, out_hbm, didx, ones_v, acc):
        cid = lax.axis_index("c")
        sid = lax.axis_index("s")
        wid = sid * 2 + cid
        pltpu.sync_copy(zeros_hbm.at[pl.ds(0, ZR)], acc.at[pl.ds(sid * ZR, ZR)])
        pltpu.sync_copy(dstw_hbm.at[wid], didx)
        pltpu.sync_copy(ones_hbm, ones_v)
        plsc.subcore_barrier()

        @pl.loop(0, nch)
        def _(j):
            pltpu.sync_copy(ones_v, acc.at[didx.at[j]], add=True)

        plsc.subcore_barrier()
        pltpu.sync_copy(acc.at[pl.ds(sid * ZR, ZR)],
                        out_hbm.at[cid, pl.ds(sid * ZR, ZR)])

    zeros = jnp.zeros((ZR, DEGW), _f32)
    ones = jnp.ones((CH, DEGW), _f32)
    return k(dstw, zeros, ones)


def _sc_prop(xs, srcw, dstw):
    nch = srcw.shape[1]
    f = xs.shape[1]

    @functools.partial(
        pl.kernel, mesh=_sc_mesh(),
        out_type=jax.ShapeDtypeStruct((2, NPAD, f), _f32),
        scratch_types=[
            pltpu.VMEM((nch, CH), jnp.int32),
            pltpu.VMEM((nch, CH), jnp.int32),
            pltpu.VMEM((CH, f), _f32),
            pltpu.VMEM((CH, f), _f32),
            pltpu.VMEM_SHARED((NPAD, f), _f32),
            pltpu.SemaphoreType.DMA,
            pltpu.SemaphoreType.DMA,
        ],
    )
    def k(xs_hbm, srcw_hbm, dstw_hbm, zeros_hbm, out_hbm,
          sidx, didx, rows0, rows1, acc, sem0, sem1):
        cid = lax.axis_index("c")
        sid = lax.axis_index("s")
        wid = sid * 2 + cid
        pltpu.sync_copy(zeros_hbm.at[pl.ds(0, ZR)], acc.at[pl.ds(sid * ZR, ZR)])
        pltpu.sync_copy(srcw_hbm.at[wid], sidx)
        pltpu.sync_copy(dstw_hbm.at[wid], didx)
        plsc.subcore_barrier()

        pltpu.async_copy(xs_hbm.at[sidx.at[0]], rows0, sem0)

        @pl.loop(0, nch, step=2)
        def _(j):
            pltpu.async_copy(xs_hbm.at[sidx.at[j + 1]], rows1, sem1)
            pltpu.make_async_copy(xs_hbm.at[sidx.at[j]], rows0, sem0).wait()
            pltpu.sync_copy(rows0, acc.at[didx.at[j]], add=True)

            @pl.when(j + 2 < nch)
            def _():
                pltpu.async_copy(xs_hbm.at[sidx.at[j + 2]], rows0, sem0)

            pltpu.make_async_copy(xs_hbm.at[sidx.at[j + 1]], rows1, sem1).wait()
            pltpu.sync_copy(rows1, acc.at[didx.at[j + 1]], add=True)

        plsc.subcore_barrier()
        pltpu.sync_copy(acc.at[pl.ds(sid * ZR, ZR)],
                        out_hbm.at[cid, pl.ds(sid * ZR, ZR)])

    zeros = jnp.zeros((ZR, f), _f32)
    return k(xs, srcw, dstw, zeros)


def _sc_zgather(z, srcw, dstw):
    nch = srcw.shape[1]
    epad = NW * nch * CH
    epw = nch * CH
    out_sds = jax.ShapeDtypeStruct((epad, ZDIM), _f32)

    @functools.partial(
        pl.kernel, mesh=_sc_mesh(), compiler_params=_linear_cp,
        out_type=(out_sds, out_sds),
        scratch_types=[
            pltpu.VMEM((nch, CH), jnp.int32),
            pltpu.VMEM((nch, CH), jnp.int32),
            pltpu.VMEM((CH, ZDIM), _f32),
            pltpu.VMEM((CH, ZDIM), _f32),
            pltpu.VMEM((CH, ZDIM), _f32),
            pltpu.VMEM((CH, ZDIM), _f32),
            pltpu.SemaphoreType.DMA,
            pltpu.SemaphoreType.DMA,
        ],
    )
    def k(z_hbm, srcw_hbm, dstw_hbm, zs_hbm, zd_hbm,
          sidx, didx, s0, d0, s1, d1, sem0, sem1):
        cid = lax.axis_index("c")
        sid = lax.axis_index("s")
        wid = sid * 2 + cid
        pltpu.sync_copy(srcw_hbm.at[wid], sidx)
        pltpu.sync_copy(dstw_hbm.at[wid], didx)

        pltpu.async_copy(z_hbm.at[sidx.at[0]], s0, sem0)
        pltpu.async_copy(z_hbm.at[didx.at[0]], d0, sem0)

        @pl.loop(0, nch, step=2)
        def _(j):
            pltpu.async_copy(z_hbm.at[sidx.at[j + 1]], s1, sem1)
            pltpu.async_copy(z_hbm.at[didx.at[j + 1]], d1, sem1)
            base = wid * epw + j * CH
            pltpu.make_async_copy(z_hbm.at[sidx.at[j]], s0, sem0).wait()
            pltpu.make_async_copy(z_hbm.at[didx.at[j]], d0, sem0).wait()
            pltpu.sync_copy(s0, zs_hbm.at[pl.ds(base, CH)])
            pltpu.sync_copy(d0, zd_hbm.at[pl.ds(base, CH)])

            @pl.when(j + 2 < nch)
            def _():
                pltpu.async_copy(z_hbm.at[sidx.at[j + 2]], s0, sem0)
                pltpu.async_copy(z_hbm.at[didx.at[j + 2]], d0, sem0)

            pltpu.make_async_copy(z_hbm.at[sidx.at[j + 1]], s1, sem1).wait()
            pltpu.make_async_copy(z_hbm.at[didx.at[j + 1]], d1, sem1).wait()
            pltpu.sync_copy(s1, zs_hbm.at[pl.ds(base + CH, CH)])
            pltpu.sync_copy(d1, zd_hbm.at[pl.ds(base + CH, CH)])

    return k(z, srcw, dstw)


_RB = 1000


def _full(shape):
    return pl.BlockSpec(shape, lambda i: tuple(0 for _ in shape))


def _rows(shape):
    return pl.BlockSpec(shape, lambda i: (i,) + tuple(0 for _ in shape[1:]))


def _rows3(shape):
    return pl.BlockSpec(shape, lambda i: (0, i, 0))


def _tc_a0(x1, x2, w1, w2):
    def body(x1_ref, x2_ref, w1_ref, w2_ref, o1_ref, o2_ref):
        o1_ref[...] = jnp.dot(x1_ref[...], w1_ref[...],
                              preferred_element_type=_f32)
        o2_ref[...] = jnp.dot(x2_ref[...], w2_ref[...],
                              preferred_element_type=_f32)

    d = x1.shape[1]
    h = w1.shape[1]
    return pl.pallas_call(
        body,
        grid=(N // _RB,),
        in_specs=[_rows((_RB, d)), _rows((_RB, d)), _full((d, h)), _full((d, h))],
        out_specs=[_rows((_RB, h)), _rows((_RB, h))],
        out_shape=[jax.ShapeDtypeStruct((N, h), _f32)] * 2,
    )(x1, x2, w1, w2)


def _tc_a1(degp, xw1, xw2):
    def body(degp_ref, xw1_ref, xw2_ref, xs1_ref, xs2_ref, dinv_ref):
        dp = degp_ref[...]
        deg = dp[0] + dp[1] + 1.0
        dinv = 1.0 / jnp.sqrt(deg)
        dinv_ref[...] = dinv
        d1 = dinv[:, 0:1]
        xs1_ref[...] = xw1_ref[...] * d1
        xs2_ref[...] = xw2_ref[...] * d1

    h = xw1.shape[1]
    return pl.pallas_call(
        body,
        grid=(N // _RB,),
        in_specs=[_rows3((2, _RB, DEGW)), _rows((_RB, h)), _rows((_RB, h))],
        out_specs=[_rows((_RB, h)), _rows((_RB, h)), _rows((_RB, DEGW))],
        out_shape=[jax.ShapeDtypeStruct((N, h), _f32),
                   jax.ShapeDtypeStruct((N, h), _f32),
                   jax.ShapeDtypeStruct((N, DEGW), _f32)],
    )(degp, xw1, xw2)


def _tc_b(acc1, acc2, xs1, xs2, dinv16, w2a, w2b, b1a, b1b):
    def body(a1_ref, a2_ref, xs1_ref, xs2_ref, dinv_ref,
             w2a_ref, w2b_ref, b1a_ref, b1b_ref, out_ref):
        dinv = dinv_ref[:, 0:1]
        a1 = a1_ref[...]
        a2 = a2_ref[...]
        h1 = jax.nn.relu(dinv * (a1[0] + a1[1] + xs1_ref[...]) + b1a_ref[...])
        h2 = jax.nn.relu(dinv * (a2[0] + a2[1] + xs2_ref[...]) + b1b_ref[...])
        xw1 = jnp.dot(h1, w2a_ref[...], preferred_element_type=_f32)
        xw2 = jnp.dot(h2, w2b_ref[...], preferred_element_type=_f32)
        out_ref[...] = jnp.concatenate([xw1 * dinv, xw2 * dinv], axis=1)

    h1w = xs1.shape[1]
    h2w = w2a.shape[1]
    return pl.pallas_call(
        body,
        grid=(N // _RB,),
        in_specs=[_rows3((2, _RB, h1w)), _rows3((2, _RB, h1w)),
                  _rows((_RB, h1w)), _rows((_RB, h1w)), _rows((_RB, DEGW)),
                  _full((h1w, h2w)), _full((h1w, h2w)),
                  _full((1, h1w)), _full((1, h1w))],
        out_specs=[_rows((_RB, 2 * h2w))],
        out_shape=[jax.ShapeDtypeStruct((N, 2 * h2w), _f32)],
    )(acc1, acc2, xs1, xs2, dinv16, w2a, w2b, b1a, b1b)[0]


def _tc_c1(acc3, xscat, dinv16, b2cat, fw, fb, muw, mub, lvw, lvb, eps):
    def body(a_ref, xs_ref, dinv_ref, b2_ref, fw_ref, fb_ref,
             muw_ref, mub_ref, lvw_ref, lvb_ref, eps_ref,
             z_ref, mu_ref, lv_ref):
        dinv = dinv_ref[:, 0:1]
        a = a_ref[...]
        hcat = jax.nn.relu(dinv * (a[0] + a[1] + xs_ref[...]) + b2_ref[...])
        h = jax.nn.relu(jnp.dot(hcat, fw_ref[...],
                                preferred_element_type=_f32) + fb_ref[...])
        mu = jnp.dot(h, muw_ref[...], preferred_element_type=_f32) + mub_ref[...]
        lv = jnp.dot(h, lvw_ref[...], preferred_element_type=_f32) + lvb_ref[...]
        mu_ref[...] = mu
        lv_ref[...] = lv
        z_ref[...] = mu + eps_ref[...] * jnp.exp(0.5 * lv)

    cw = xscat.shape[1]
    fu = fw.shape[1]
    return pl.pallas_call(
        body,
        grid=(N // _RB,),
        in_specs=[_rows3((2, _RB, cw)), _rows((_RB, cw)), _rows((_RB, DEGW)),
                  _full((1, cw)), _full((cw, fu)), _full((1, fu)),
                  _full((fu, ZDIM)), _full((1, ZDIM)),
                  _full((fu, ZDIM)), _full((1, ZDIM)),
                  _rows((_RB, ZDIM))],
        out_specs=[_rows((_RB, ZDIM))] * 3,
        out_shape=[jax.ShapeDtypeStruct((N, ZDIM), _f32)] * 3,
    )(acc3, xscat, dinv16, b2cat, fw, fb, muw, mub, lvw, lvb, eps)


def _tc_c2(z, w11, b11, w12, b12, w21, b21, w22, b22, ws1, bs1, ws2, bs2):
    def body(z_ref, w11_ref, b11_ref, w12_ref, b12_ref,
             w21_ref, b21_ref, w22_ref, b22_ref,
             ws1_ref, bs1_ref, ws2_ref, bs2_ref,
             x1_ref, x2_ref, xs_ref):
        z = z_ref[...]
        t1 = jax.nn.relu(jnp.dot(z, w11_ref[...],
                                 preferred_element_type=_f32) + b11_ref[...])
        x1_ref[...] = jnp.dot(t1, w12_ref[...],
                              preferred_element_type=_f32) + b12_ref[...]
        t2 = jax.nn.relu(jnp.dot(z, w21_ref[...],
                                 preferred_element_type=_f32) + b21_ref[...])
        x2_ref[...] = jnp.dot(t2, w22_ref[...],
                              preferred_element_type=_f32) + b22_ref[...]
        ts = jax.nn.relu(jnp.dot(z, ws1_ref[...],
                                 preferred_element_type=_f32) + bs1_ref[...])
        xs_ref[...] = jnp.dot(ts, ws2_ref[...],
                              preferred_element_type=_f32) + bs2_ref[...]

    fu = w11.shape[1]
    r1 = w12.shape[1]
    r2 = w22.shape[1]
    rs = ws2.shape[1]
    return pl.pallas_call(
        body,
        grid=(N // _RB,),
        in_specs=[_rows((_RB, ZDIM)),
                  _full((ZDIM, fu)), _full((1, fu)), _full((fu, r1)), _full((1, r1)),
                  _full((ZDIM, fu)), _full((1, fu)), _full((fu, r2)), _full((1, r2)),
                  _full((ZDIM, fu)), _full((1, fu)), _full((fu, rs)), _full((1, rs))],
        out_specs=[_rows((_RB, r1)), _rows((_RB, r2)), _rows((_RB, rs))],
        out_shape=[jax.ShapeDtypeStruct((N, r1), _f32),
                   jax.ShapeDtypeStruct((N, r2), _f32),
                   jax.ShapeDtypeStruct((N, rs), _f32)],
    )(z, w11, b11, w12, b12, w21, b21, w22, b22, ws1, bs1, ws2, bs2)


def _tc_d(zs, zd):
    eb = 2048

    def body(zs_ref, zd_ref, o_ref):
        prod = zs_ref[...] * zd_ref[...]
        s = jnp.dot(prod, jnp.ones((ZDIM, 1), _f32),
                    preferred_element_type=_f32)
        o_ref[...] = jax.nn.sigmoid(s)

    epad = zs.shape[0]
    return pl.pallas_call(
        body,
        grid=(epad // eb,),
        in_specs=[_rows((eb, ZDIM)), _rows((eb, ZDIM))],
        out_specs=[_rows((eb, 1))],
        out_shape=[jax.ShapeDtypeStruct((epad, 1), _f32)],
    )(zs, zd)[0]


def kernel(x_omics1, x_omics2, edge_index, params):
    p = params
    src = edge_index[0].astype(jnp.int32)
    dst = edge_index[1].astype(jnp.int32)
    e = src.shape[0]
    epad = -(-e // (NW * CH)) * (NW * CH)
    nch = epad // (NW * CH)
    pad = epad - e
    srcw = jnp.concatenate([src, jnp.zeros((pad,), jnp.int32)]).reshape(NW, nch, CH)
    dstw_sink = jnp.concatenate(
        [dst, jnp.full((pad,), SINK, jnp.int32)]).reshape(NW, nch, CH)
    dstw_zero = jnp.concatenate(
        [dst, jnp.zeros((pad,), jnp.int32)]).reshape(NW, nch, CH)

    degp = _sc_degree(dstw_sink)
    xw1, xw2 = _tc_a0(x_omics1, x_omics2, p['gcn1_w1'], p['gcn2_w1'])
    xs1, xs2, dinv16 = _tc_a1(degp, xw1, xw2)

    acc1 = _sc_prop(xs1, srcw, dstw_sink)
    acc2 = _sc_prop(xs2, srcw, dstw_sink)

    xscat = _tc_b(acc1, acc2, xs1, xs2, dinv16,
                  p['gcn1_w2'], p['gcn2_w2'],
                  p['gcn1_b1'][None, :], p['gcn2_b1'][None, :])

    acc3 = _sc_prop(xscat, srcw, dstw_sink)

    b2cat = jnp.concatenate([p['gcn1_b2'], p['gcn2_b2']])[None, :]
    eps = jax.random.normal(jax.random.key(42), (N, ZDIM), _f32)
    z, mu, logvar = _tc_c1(acc3, xscat, dinv16, b2cat,
                           p['fuse_w'], p['fuse_b'][None, :],
                           p['mu_w'], p['mu_b'][None, :],
                           p['logvar_w'], p['logvar_b'][None, :], eps)

    xhat1, xhat2, xhat_s = _tc_c2(
        z, p['rec1_w1'], p['rec1_b1'][None, :], p['rec1_w2'], p['rec1_b2'][None, :],
        p['rec2_w1'], p['rec2_b1'][None, :], p['rec2_w2'], p['rec2_b2'][None, :],
        p['recs_w1'], p['recs_b1'][None, :], p['recs_w2'], p['recs_b2'][None, :])

    zs, zd = _sc_zgather(z, srcw, dstw_zero)
    adj = _tc_d(zs, zd)
    adj_pred = adj[:e, 0]

    return (z, mu, logvar, xhat1, xhat2, xhat_s, adj_pred)
